# baseline (speedup 1.0000x reference)
import functools

import jax
import jax.numpy as jnp
import numpy as np
from jax import lax
from jax.experimental import pallas as pl
from jax.experimental.pallas import tpu as pltpu
from jax.experimental.pallas import tpu_sc as plsc

_OBJ0, _OBJ1 = 256, 4096
_K = 410
_POOL = 16
_BLK = 512
_THRESHOLD = 0.1
_LAMDA = 1.0

_NW = 32
_SC_ROWS = 128
_TC_ROWS = _OBJ0 - _SC_ROWS
_SC_RPW = _SC_ROWS // _NW
_NV = _OBJ1 // 16

_A = np.zeros((_OBJ1, _OBJ1 // _POOL), dtype=np.float32)
_A[np.arange(_OBJ1), np.arange(_OBJ1) // _POOL] = 1.0 / 32.0


def _pool_body(w1_ref, w2_ref, a_ref, out_ref):
    s = w1_ref[...] + w2_ref[...]
    out_ref[...] = jax.lax.dot(s, a_ref[...], preferred_element_type=jnp.float32)


def _splat(x):
    return jnp.full((16,), x)


def _tc_select_body(e_ref, sig_ref, loss_ref):
    e = e_ref[...]
    n = e.shape[0]
    bits = jax.lax.bitcast_convert_type(jnp.abs(e), jnp.int32)

    lo = jnp.zeros((n, 1), jnp.int32)
    hi = jnp.full((n, 1), 0x7F800000, jnp.int32)

    def body(_, carry):
        lo, hi = carry
        mid = lo + ((hi - lo) >> 1)
        cnt = jnp.sum((bits <= mid).astype(jnp.int32), axis=1, keepdims=True)
        take = cnt >= _K
        return jnp.where(take, lo, mid + 1), jnp.where(take, mid, hi)

    lo, hi = jax.lax.fori_loop(0, 31, body, (lo, hi))
    kth = lo

    less = bits < kth
    eq = bits == kth
    cnt_less = jnp.sum(less.astype(jnp.float32), axis=1, keepdims=True)
    cnt_eq = jnp.sum(eq.astype(jnp.float32), axis=1, keepdims=True)
    sum_less = jnp.sum(jnp.where(less, e, 0.0), axis=1, keepdims=True)
    sum_eq = jnp.sum(jnp.where(eq, e, 0.0), axis=1, keepdims=True)
    total = jnp.sum(e, axis=1, keepdims=True)

    need = jnp.float32(_K) - cnt_less
    bottom = sum_less + sum_eq * need / cnt_eq
    pred = (total - bottom) / jnp.float32(_OBJ1)

    sig = sig_ref[...]
    loss_ref[0, 0] = jnp.sum(jnp.maximum(_THRESHOLD - sig * pred, 0.0))


def _sc_select_body(ext_hbm, sig_hbm, out_hbm, row_all, comp_bits, comp_val,
                    hist, sig_v, out_v, sem):
    wid = lax.axis_index("s") * 2 + lax.axis_index("c")
    row0 = wid * _SC_RPW

    iota = lax.broadcasted_iota(jnp.int32, (16,), 0)
    iota256 = iota * 256
    ones_i = jnp.ones((16,), jnp.int32)
    zeros_f = jnp.zeros((16,), jnp.float32)
    zeros_i = jnp.zeros((16,), jnp.int32)

    pltpu.async_copy(ext_hbm.at[pl.ds(row0, _SC_RPW)], row_all, sem).wait()
    pltpu.sync_copy(sig_hbm, sig_v.at[pl.ds(0, _OBJ0)])

    def row_body(r, pred_vec):
        def zero_body(j, _):
            for t in range(16):
                hist[pl.ds(j * 256 + t * 16, 16)] = zeros_i
            return 0

        lax.fori_loop(0, 16, zero_body, 0, unroll=True)

        def hist_body(j, tot):
            for t in range(8):
                v = row_all[r, pl.ds(j * 128 + t * 16, 16)]
                b = lax.bitcast_convert_type(v, jnp.int32) & 0x7FFFFFFF
                bucket = b >> 23
                plsc.addupdate_scatter(hist, [bucket + iota256], ones_i)
                tot = tot + v
            return tot

        tot_vec = lax.fori_loop(0, _NV // 8, hist_body, zeros_f)
        total = jnp.sum(tot_vec)

        cum = zeros_i
        b_vec = zeros_i
        below_vec = zeros_i
        for t in range(16):
            h = hist[pl.ds(t * 16, 16)]
            for l in range(1, 16):
                h = h + hist[pl.ds(l * 256 + t * 16, 16)]
            cs = plsc.cumsum(h) + cum
            hit = cs < _K
            b_vec = b_vec + plsc.all_reduce_population_count(hit)
            below_vec = jnp.maximum(below_vec, jnp.where(hit, cs, zeros_i))
            cum = _splat(jnp.max(cs))
        below = jnp.max(below_vec)
        need = _K - below

        def comp_body(j, carry):
            off, sum_lt = carry
            for t in range(4):
                v = row_all[r, pl.ds(j * 64 + t * 16, 16)]
                b = lax.bitcast_convert_type(v, jnp.int32) & 0x7FFFFFFF
                bucket = b >> 23
                lt = bucket < b_vec
                eq = bucket == b_vec
                sum_lt = sum_lt + jnp.where(lt, v, zeros_f)
                idx = off + plsc.cumsum(eq.astype(jnp.int32)) - 1
                plsc.store_scatter(comp_bits, [idx], b, mask=eq)
                plsc.store_scatter(comp_val, [idx], v, mask=eq)
                off = off + plsc.all_reduce_population_count(eq)
            return off, sum_lt

        off_vec, sum_lt_vec = lax.fori_loop(0, _NV // 4, comp_body,
                                            (zeros_i, zeros_f))
        sum_lt = jnp.sum(sum_lt_vec)
        m = jnp.max(off_vec)
        sent = jnp.full((16,), 0x7FFFFFFF, jnp.int32)
        for t in range(4):
            plsc.store_scatter(comp_bits, [off_vec + iota + t * 16], sent)
        nv4 = (m + 63) >> 6

        lo0 = b_vec << 23
        hi0 = lo0 + ((1 << 23) - 1)

        def bs_body(_, carry):
            lo, hi = carry
            mid = lo + ((hi - lo) >> 1)

            def cnt_body(i, c):
                for t in range(4):
                    b = comp_bits[pl.ds(i * 64 + t * 16, 16)]
                    c = c + plsc.all_reduce_population_count(b <= mid)
                return c

            cnt = lax.fori_loop(0, nv4, cnt_body, zeros_i)
            take = cnt >= need
            return jnp.where(take, lo, mid + 1), jnp.where(take, mid, hi)

        kth, _ = lax.fori_loop(0, 23, bs_body, (lo0, hi0))

        def fin_body(i, carry):
            s_less, c_less, s_eq, c_eq = carry
            for t in range(4):
                b = comp_bits[pl.ds(i * 64 + t * 16, 16)]
                v = comp_val[pl.ds(i * 64 + t * 16, 16)]
                less = b < kth
                eq = b == kth
                s_less = s_less + jnp.where(less, v, zeros_f)
                c_less = c_less + plsc.all_reduce_population_count(less)
                s_eq = s_eq + jnp.where(eq, v, zeros_f)
                c_eq = c_eq + plsc.all_reduce_population_count(eq)
            return s_less, c_less, s_eq, c_eq

        s_less_v, c_less_v, s_eq_v, c_eq_v = lax.fori_loop(
            0, nv4, fin_body, (zeros_f, zeros_i, zeros_f, zeros_i))
        s_less = jnp.sum(s_less_v)
        s_eq = jnp.sum(s_eq_v)

        take_eq = (need - c_less_v).astype(jnp.float32)
        bottom_v = (sum_lt + s_less) + take_eq * s_eq / c_eq_v.astype(jnp.float32)
        pred_v = (total - bottom_v) / jnp.float32(_OBJ1)
        return jnp.where(iota == r, pred_v, pred_vec)

    pred_vec = lax.fori_loop(0, _SC_RPW, row_body, zeros_f)

    sig16 = sig_v[pl.ds(_TC_ROWS + row0, 16)]
    hinge = jnp.where(iota < _SC_RPW,
                      jnp.maximum(_THRESHOLD - sig16 * pred_vec, 0.0),
                      zeros_f)
    out_v[...] = hinge
    pltpu.sync_copy(out_v, out_hbm.at[wid])


def _sc_select(extraction, sig):
    mesh = plsc.VectorSubcoreMesh(core_axis_name="c", subcore_axis_name="s")
    return pl.kernel(
        _sc_select_body,
        out_type=jax.ShapeDtypeStruct((_NW, 16), jnp.float32),
        mesh=mesh,
        compiler_params=pltpu.CompilerParams(needs_layout_passes=False),
        scratch_types=[
            pltpu.VMEM((_SC_RPW, _OBJ1), jnp.float32),
            pltpu.VMEM((_OBJ1 + 64,), jnp.int32),
            pltpu.VMEM((_OBJ1 + 64,), jnp.float32),
            pltpu.VMEM((_OBJ1,), jnp.int32),
            pltpu.VMEM((_OBJ0 + 16,), jnp.float32),
            pltpu.VMEM((16,), jnp.float32),
            pltpu.SemaphoreType.DMA,
        ],
    )(extraction, sig)


def _final_body(scp_ref, tcp_ref, out_ref):
    out_ref[0, 0] = _LAMDA * (jnp.sum(scp_ref[...]) + tcp_ref[0, 0])


def _pool_half(weight1, weight2, a, first_block):
    n_blocks = (_OBJ0 // 2) * _POOL // _BLK
    return pl.pallas_call(
        _pool_body,
        grid=(n_blocks,),
        in_specs=[
            pl.BlockSpec((_BLK, _OBJ1), lambda i: (i + first_block, 0)),
            pl.BlockSpec((_BLK, _OBJ1), lambda i: (i + first_block, 0)),
            pl.BlockSpec((_OBJ1, _OBJ0), lambda i: (0, 0)),
        ],
        out_specs=pl.BlockSpec((_BLK, _OBJ0), lambda i: (i, 0)),
        out_shape=jax.ShapeDtypeStruct((n_blocks * _BLK, _OBJ0), jnp.float32),
    )(weight1, weight2, a)


@jax.jit
def kernel(weight1, weight2, sig):
    a = jnp.asarray(_A)
    sig_col = sig.reshape(_OBJ0, 1)

    pooled_sc = _pool_half(weight1, weight2, a, _TC_ROWS * _POOL // _BLK)
    extraction_sc = pooled_sc.reshape(_SC_ROWS, _OBJ1)
    sc_partials = _sc_select(extraction_sc, sig)

    pooled_tc = _pool_half(weight1, weight2, a, 0)
    extraction_tc = pooled_tc.reshape(_TC_ROWS, _OBJ1)

    tc_partial = pl.pallas_call(
        _tc_select_body,
        grid=(1,),
        in_specs=[
            pl.BlockSpec((_TC_ROWS, _OBJ1), lambda i: (0, 0)),
            pl.BlockSpec((_TC_ROWS, 1), lambda i: (0, 0)),
        ],
        out_specs=pl.BlockSpec((1, 1), lambda i: (0, 0), memory_space=pltpu.SMEM),
        out_shape=jax.ShapeDtypeStruct((1, 1), jnp.float32),
    )(extraction_tc, sig_col)

    loss = pl.pallas_call(
        _final_body,
        in_specs=[
            pl.BlockSpec((_NW, 16), lambda: (0, 0)),
            pl.BlockSpec(memory_space=pltpu.SMEM),
        ],
        out_specs=pl.BlockSpec(memory_space=pltpu.SMEM),
        out_shape=jax.ShapeDtypeStruct((1, 1), jnp.float32),
    )(sc_partials, tc_partial)
    return loss[0, 0]


# speedup vs baseline: 1.0896x; 1.0896x over previous
import functools

import jax
import jax.numpy as jnp
import numpy as np
from jax import lax
from jax.experimental import pallas as pl
from jax.experimental.pallas import tpu as pltpu
from jax.experimental.pallas import tpu_sc as plsc

_OBJ0, _OBJ1 = 256, 4096
_K = 410
_POOL = 16
_BLK = 256
_THRESHOLD = 0.1
_LAMDA = 1.0

_NW = 32
_SC_ROWS = 128
_TC_ROWS = _OBJ0 - _SC_ROWS
_SC_RPW = _SC_ROWS // _NW
_NV = _OBJ1 // 16

_A = np.zeros((_OBJ1, _OBJ1 // _POOL), dtype=np.float32)
_A[np.arange(_OBJ1), np.arange(_OBJ1) // _POOL] = 1.0 / 32.0


def _pool_body(w1_ref, w2_ref, a_ref, out_ref):
    s = w1_ref[...] + w2_ref[...]
    p = jax.lax.dot(s, a_ref[...], preferred_element_type=jnp.float32)
    out_ref[...] = p.reshape(_BLK // _POOL, _OBJ1)


def _splat(x):
    return jnp.full((16,), x)


def _tc_select_body(e_ref, sig_ref, loss_ref):
    e = e_ref[...]
    n = e.shape[0]
    bits = jax.lax.bitcast_convert_type(jnp.abs(e), jnp.int32)

    lo = jnp.zeros((n, 1), jnp.int32)
    hi = jnp.full((n, 1), 0x7F800000, jnp.int32)

    def body(_, carry):
        lo, hi = carry
        mid = lo + ((hi - lo) >> 1)
        cnt = jnp.sum((bits <= mid).astype(jnp.int32), axis=1, keepdims=True)
        take = cnt >= _K
        return jnp.where(take, lo, mid + 1), jnp.where(take, mid, hi)

    lo, hi = jax.lax.fori_loop(0, 31, body, (lo, hi))
    kth = lo

    less = bits < kth
    eq = bits == kth
    cnt_less = jnp.sum(less.astype(jnp.float32), axis=1, keepdims=True)
    cnt_eq = jnp.sum(eq.astype(jnp.float32), axis=1, keepdims=True)
    sum_less = jnp.sum(jnp.where(less, e, 0.0), axis=1, keepdims=True)
    sum_eq = jnp.sum(jnp.where(eq, e, 0.0), axis=1, keepdims=True)
    total = jnp.sum(e, axis=1, keepdims=True)

    need = jnp.float32(_K) - cnt_less
    bottom = sum_less + sum_eq * need / cnt_eq
    pred = (total - bottom) / jnp.float32(_OBJ1)

    sig = sig_ref[...]
    loss_ref[0, 0] = jnp.sum(jnp.maximum(_THRESHOLD - sig * pred, 0.0))


def _sc_select_body(ext_hbm, sig_hbm, out_hbm, row_all, comp_bits, comp_val,
                    hist, sig_v, out_v, sem):
    wid = lax.axis_index("s") * 2 + lax.axis_index("c")
    row0 = wid * _SC_RPW

    iota = lax.broadcasted_iota(jnp.int32, (16,), 0)
    iota256 = iota * 256
    ones_i = jnp.ones((16,), jnp.int32)
    zeros_f = jnp.zeros((16,), jnp.float32)
    zeros_i = jnp.zeros((16,), jnp.int32)

    pltpu.async_copy(ext_hbm.at[pl.ds(row0, _SC_RPW)], row_all, sem).wait()
    pltpu.sync_copy(sig_hbm, sig_v.at[pl.ds(0, _OBJ0)])

    def row_body(r, pred_vec):
        def zero_body(j, _):
            for t in range(16):
                hist[pl.ds(j * 256 + t * 16, 16)] = zeros_i
            return 0

        lax.fori_loop(0, 16, zero_body, 0, unroll=True)

        def hist_body(j, tot):
            for t in range(8):
                v = row_all[r, pl.ds(j * 128 + t * 16, 16)]
                b = lax.bitcast_convert_type(v, jnp.int32) & 0x7FFFFFFF
                bucket = b >> 23
                plsc.addupdate_scatter(hist, [bucket + iota256], ones_i)
                tot = tot + v
            return tot

        tot_vec = lax.fori_loop(0, _NV // 8, hist_body, zeros_f)
        total = jnp.sum(tot_vec)

        cum = zeros_i
        b_vec = zeros_i
        below_vec = zeros_i
        for t in range(16):
            h = hist[pl.ds(t * 16, 16)]
            for l in range(1, 16):
                h = h + hist[pl.ds(l * 256 + t * 16, 16)]
            cs = plsc.cumsum(h) + cum
            hit = cs < _K
            b_vec = b_vec + plsc.all_reduce_population_count(hit)
            below_vec = jnp.maximum(below_vec, jnp.where(hit, cs, zeros_i))
            cum = _splat(jnp.max(cs))
        below = jnp.max(below_vec)
        need = _K - below

        def comp_body(j, carry):
            off, sum_lt = carry
            for t in range(4):
                v = row_all[r, pl.ds(j * 64 + t * 16, 16)]
                b = lax.bitcast_convert_type(v, jnp.int32) & 0x7FFFFFFF
                bucket = b >> 23
                lt = bucket < b_vec
                eq = bucket == b_vec
                sum_lt = sum_lt + jnp.where(lt, v, zeros_f)
                idx = off + plsc.cumsum(eq.astype(jnp.int32)) - 1
                plsc.store_scatter(comp_bits, [idx], b, mask=eq)
                plsc.store_scatter(comp_val, [idx], v, mask=eq)
                off = off + plsc.all_reduce_population_count(eq)
            return off, sum_lt

        off_vec, sum_lt_vec = lax.fori_loop(0, _NV // 4, comp_body,
                                            (zeros_i, zeros_f))
        sum_lt = jnp.sum(sum_lt_vec)
        m = jnp.max(off_vec)
        sent = jnp.full((16,), 0x7FFFFFFF, jnp.int32)
        for t in range(4):
            plsc.store_scatter(comp_bits, [off_vec + iota + t * 16], sent)
        nv4 = (m + 63) >> 6

        lo0 = b_vec << 23
        hi0 = lo0 + ((1 << 23) - 1)

        def bs_body(_, carry):
            lo, hi = carry
            mid = lo + ((hi - lo) >> 1)

            def cnt_body(i, c):
                for t in range(4):
                    b = comp_bits[pl.ds(i * 64 + t * 16, 16)]
                    c = c + plsc.all_reduce_population_count(b <= mid)
                return c

            cnt = lax.fori_loop(0, nv4, cnt_body, zeros_i)
            take = cnt >= need
            return jnp.where(take, lo, mid + 1), jnp.where(take, mid, hi)

        kth, _ = lax.fori_loop(0, 23, bs_body, (lo0, hi0))

        def fin_body(i, carry):
            s_less, c_less, s_eq, c_eq = carry
            for t in range(4):
                b = comp_bits[pl.ds(i * 64 + t * 16, 16)]
                v = comp_val[pl.ds(i * 64 + t * 16, 16)]
                less = b < kth
                eq = b == kth
                s_less = s_less + jnp.where(less, v, zeros_f)
                c_less = c_less + plsc.all_reduce_population_count(less)
                s_eq = s_eq + jnp.where(eq, v, zeros_f)
                c_eq = c_eq + plsc.all_reduce_population_count(eq)
            return s_less, c_less, s_eq, c_eq

        s_less_v, c_less_v, s_eq_v, c_eq_v = lax.fori_loop(
            0, nv4, fin_body, (zeros_f, zeros_i, zeros_f, zeros_i))
        s_less = jnp.sum(s_less_v)
        s_eq = jnp.sum(s_eq_v)

        take_eq = (need - c_less_v).astype(jnp.float32)
        bottom_v = (sum_lt + s_less) + take_eq * s_eq / c_eq_v.astype(jnp.float32)
        pred_v = (total - bottom_v) / jnp.float32(_OBJ1)
        return jnp.where(iota == r, pred_v, pred_vec)

    pred_vec = lax.fori_loop(0, _SC_RPW, row_body, zeros_f)

    sig16 = sig_v[pl.ds(_TC_ROWS + row0, 16)]
    hinge = jnp.where(iota < _SC_RPW,
                      jnp.maximum(_THRESHOLD - sig16 * pred_vec, 0.0),
                      zeros_f)
    out_v[...] = hinge
    pltpu.sync_copy(out_v, out_hbm.at[wid])


def _sc_select(extraction, sig):
    mesh = plsc.VectorSubcoreMesh(core_axis_name="c", subcore_axis_name="s")
    return pl.kernel(
        _sc_select_body,
        out_type=jax.ShapeDtypeStruct((_NW, 16), jnp.float32),
        mesh=mesh,
        compiler_params=pltpu.CompilerParams(needs_layout_passes=False),
        scratch_types=[
            pltpu.VMEM((_SC_RPW, _OBJ1), jnp.float32),
            pltpu.VMEM((_OBJ1 + 64,), jnp.int32),
            pltpu.VMEM((_OBJ1 + 64,), jnp.float32),
            pltpu.VMEM((_OBJ1,), jnp.int32),
            pltpu.VMEM((_OBJ0 + 16,), jnp.float32),
            pltpu.VMEM((16,), jnp.float32),
            pltpu.SemaphoreType.DMA,
        ],
    )(extraction, sig)


def _final_body(scp_ref, tcp_ref, out_ref):
    out_ref[0, 0] = _LAMDA * (jnp.sum(scp_ref[...]) + tcp_ref[0, 0])


def _pool_half(weight1, weight2, a, first_block):
    n_blocks = (_OBJ0 // 2) * _POOL // _BLK
    return pl.pallas_call(
        _pool_body,
        grid=(n_blocks,),
        in_specs=[
            pl.BlockSpec((_BLK, _OBJ1), lambda i: (i + first_block, 0)),
            pl.BlockSpec((_BLK, _OBJ1), lambda i: (i + first_block, 0)),
            pl.BlockSpec((_OBJ1, _OBJ0), lambda i: (0, 0)),
        ],
        out_specs=pl.BlockSpec((_BLK // _POOL, _OBJ1), lambda i: (i, 0)),
        out_shape=jax.ShapeDtypeStruct(
            (n_blocks * _BLK // _POOL, _OBJ1), jnp.float32),
    )(weight1, weight2, a)


@jax.jit
def kernel(weight1, weight2, sig):
    a = jnp.asarray(_A)
    sig_col = sig.reshape(_OBJ0, 1)

    extraction_sc = _pool_half(weight1, weight2, a, _TC_ROWS * _POOL // _BLK)
    sc_partials = _sc_select(extraction_sc, sig)

    extraction_tc = _pool_half(weight1, weight2, a, 0)

    tc_partial = pl.pallas_call(
        _tc_select_body,
        grid=(1,),
        in_specs=[
            pl.BlockSpec((_TC_ROWS, _OBJ1), lambda i: (0, 0)),
            pl.BlockSpec((_TC_ROWS, 1), lambda i: (0, 0)),
        ],
        out_specs=pl.BlockSpec((1, 1), lambda i: (0, 0), memory_space=pltpu.SMEM),
        out_shape=jax.ShapeDtypeStruct((1, 1), jnp.float32),
    )(extraction_tc, sig_col)

    loss = pl.pallas_call(
        _final_body,
        in_specs=[
            pl.BlockSpec((_NW, 16), lambda: (0, 0)),
            pl.BlockSpec(memory_space=pltpu.SMEM),
        ],
        out_specs=pl.BlockSpec(memory_space=pltpu.SMEM),
        out_shape=jax.ShapeDtypeStruct((1, 1), jnp.float32),
    )(sc_partials, tc_partial)
    return loss[0, 0]
